# baseline (speedup 1.0000x reference)
import functools

import jax
import jax.numpy as jnp
from jax import lax
from jax.experimental import pallas as pl
from jax.experimental.pallas import tpu as pltpu
from jax.experimental.pallas import tpu_sc as plsc

B, C, S = 8, 1000, 8192
RATIO = 0.2
N_KEEP = int(S * RATIO)
S_BLK = 4096
S_GRID = S // S_BLK

NUM_WORKERS = 32
CHUNK = (B * S) // NUM_WORKERS
ROWS = CHUNK // 128


def _gather_body(x_hbm, y_hbm, out_hbm, y_v, idx_v, vals_v, sem):
    wid = lax.axis_index("s") * 2 + lax.axis_index("c")
    base = wid * CHUNK
    pltpu.sync_copy(y_hbm.at[pl.ds(base, CHUNK)], y_v)
    b_off = (base // S) * (C * S)
    lane = lax.iota(jnp.int32, 16)
    for r in range(ROWS):
        for c8 in range(128 // 16):
            off = r * 128 + c8 * 16
            pos = base + off + lane
            yv = y_v[pl.ds(off, 16)]
            idx = (b_off + ((yv >> 3) << 16) + (((pos >> 7) & 63) << 10)
                   + ((yv & 7) << 7) + (pos & 127))
            idx_v[r, pl.ds(c8 * 16, 16)] = idx
    copies = [pltpu.async_copy(x_hbm.at[idx_v.at[r]], vals_v.at[r], sem)
              for r in range(ROWS)]
    for cp in copies:
        cp.wait()
    pltpu.sync_copy(vals_v, out_hbm.at[wid])


def _sc_gather(x_tiles_flat, y_flat):
    mesh = plsc.VectorSubcoreMesh(core_axis_name="c", subcore_axis_name="s")
    f = functools.partial(
        pl.kernel,
        mesh=mesh,
        out_type=jax.ShapeDtypeStruct((NUM_WORKERS, ROWS, 128), jnp.float32),
        scratch_types=[
            pltpu.VMEM((CHUNK,), jnp.int32),
            pltpu.VMEM((ROWS, 128), jnp.int32),
            pltpu.VMEM((ROWS, 128), jnp.float32),
            pltpu.SemaphoreType.DMA,
        ],
    )(_gather_body)
    return f(x_tiles_flat, y_flat)


def _ce_topk_kernel(x_ref, g_ref, out_ref, l_ref):
    b = pl.program_id(0)
    sb = pl.program_id(1)

    xb = x_ref[0]
    e = jnp.exp(jnp.minimum(xb, 80.0))
    ssum = jnp.sum(e, axis=0, keepdims=True)
    l = jnp.log(ssum) - g_ref[0]
    l_ref[pl.ds(b, 1), pl.ds(sb * S_BLK, S_BLK)] = l

    @pl.when((b == B - 1) & (sb == S_GRID - 1))
    def _epilogue():
        lv = l_ref[...]
        bits = lax.bitcast_convert_type(lv, jnp.int32)
        ordv = jnp.where(bits < 0, bits ^ jnp.int32(0x7FFFFFFF), bits)
        int_min = jnp.int32(-2147483648)
        p = jnp.sum((ordv >= 0).astype(jnp.int32), axis=1, keepdims=True)
        t0 = jnp.where(p >= N_KEEP, jnp.int32(0), int_min)

        def body(i, t):
            cand = t | (jnp.int32(1) << (30 - i))
            cnt = jnp.sum((ordv >= cand).astype(jnp.int32), axis=1,
                          keepdims=True)
            return jnp.where(cnt >= N_KEEP, cand, t)

        t = lax.fori_loop(0, 31, body, t0)
        vbits = jnp.where(t < 0, t ^ jnp.int32(0x7FFFFFFF), t)
        thr = lax.bitcast_convert_type(vbits, jnp.float32)
        gt = ordv > t
        cnt_gt = jnp.sum(gt.astype(jnp.float32), axis=1, keepdims=True)
        sum_gt = jnp.sum(jnp.where(gt, lv, 0.0), axis=1, keepdims=True)
        row_total = sum_gt + (N_KEEP - cnt_gt) * thr
        out_ref[0, 0] = jnp.sum(row_total) / (N_KEEP * B)


def kernel(x, y):
    x_tiles_flat = (x.reshape(B, C // 8, 8, S // 128, 128)
                    .transpose(0, 1, 3, 2, 4).reshape(-1))
    g = _sc_gather(x_tiles_flat, y.reshape(-1).astype(jnp.int32))
    g = g.reshape(B, 1, S)
    out = pl.pallas_call(
        _ce_topk_kernel,
        grid=(B, S_GRID),
        in_specs=[
            pl.BlockSpec((1, C, S_BLK), lambda b, sb: (b, 0, sb)),
            pl.BlockSpec((1, 1, S_BLK), lambda b, sb: (b, 0, sb)),
        ],
        out_specs=pl.BlockSpec((1, 1), lambda b, sb: (0, 0),
                               memory_space=pltpu.SMEM),
        out_shape=jax.ShapeDtypeStruct((1, 1), jnp.float32),
        scratch_shapes=[pltpu.VMEM((B, S), jnp.float32)],
    )(x, g)
    return out[0, 0]


# speedup vs baseline: 1.0022x; 1.0022x over previous
import functools

import jax
import jax.numpy as jnp
from jax import lax
from jax.experimental import pallas as pl
from jax.experimental.pallas import tpu as pltpu
from jax.experimental.pallas import tpu_sc as plsc

B, C, S = 8, 1000, 8192
RATIO = 0.2
N_KEEP = int(S * RATIO)
S_BLK = 4096
S_GRID = S // S_BLK

TC_ROWS = 6
SC_ROWS = B - TC_ROWS
W_PER_ROW = 16
TOK_W = S // W_PER_ROW
CT_W = TOK_W // 128
RT = C // 8


def _sc_body(xf, y_hbm, ssum_out, g_out,
             y_v, idx_v, vals_v, ssum_v, buf_a, buf_b, sem_a, sem_b, sem_g):
    w = lax.axis_index("s") * 2 + lax.axis_index("c")
    w2 = w & 15
    rb_rel = w >> 4
    b_row = TC_ROWS + rb_rel
    s0 = w2 * TOK_W
    ct0 = w2 * CT_W

    y_off = pl.multiple_of(b_row * S + s0, TOK_W)
    pltpu.sync_copy(y_hbm.at[pl.ds(y_off, TOK_W)], y_v)
    b_off = b_row * (C * S)
    lane = lax.iota(jnp.int32, 16)
    for k in range(CT_W):
        for c8 in range(8):
            j = k * 128 + c8 * 16
            yv = y_v[pl.ds(j, 16)]
            sv = s0 + j + lane
            idx = (b_off + ((yv >> 3) << 16) + ((sv >> 7) << 10)
                   + ((yv & 7) << 7) + (sv & 127))
            idx_v[k, pl.ds(c8 * 16, 16)] = idx
    gcopies = [pltpu.async_copy(xf.at[idx_v.at[k]],
                                vals_v.at[pl.ds(k * 128, 128)], sem_g)
               for k in range(CT_W)]

    def chunk_src(c):
        off = pl.multiple_of(b_off + (c << 16) + (ct0 << 10), 1024)
        return xf.at[pl.ds(off, CT_W * 1024)]

    pltpu.async_copy(chunk_src(0), buf_a, sem_a)
    pltpu.async_copy(chunk_src(1), buf_b, sem_b)

    def accumulate(buf, acc):
        acc = list(acc)
        for ct in range(CT_W):
            for lg in range(8):
                a = acc[ct * 8 + lg]
                for r in range(8):
                    a = a + jnp.exp(buf[pl.ds(ct * 1024 + r * 128 + lg * 16,
                                              16)])
                acc[ct * 8 + lg] = a
        return acc

    def body(i2, acc):
        c0 = 2 * i2
        pltpu.make_async_copy(chunk_src(0), buf_a, sem_a).wait()

        @pl.when(c0 + 2 < RT)
        def _():
            pltpu.async_copy(chunk_src(c0 + 2), buf_a, sem_a)

        acc = accumulate(buf_a, acc)
        pltpu.make_async_copy(chunk_src(0), buf_b, sem_b).wait()

        @pl.when(c0 + 3 < RT)
        def _():
            pltpu.async_copy(chunk_src(c0 + 3), buf_b, sem_b)

        return tuple(accumulate(buf_b, acc))

    acc0 = tuple(jnp.zeros((16,), jnp.float32) for _ in range(TOK_W // 16))
    acc = lax.fori_loop(0, (RT - 1) // 2, body, acc0)
    pltpu.make_async_copy(chunk_src(0), buf_a, sem_a).wait()
    acc = accumulate(buf_a, acc)

    for k in range(TOK_W // 16):
        ssum_v[pl.ds(k * 16, 16)] = acc[k]
    s0_al = pl.multiple_of(s0, TOK_W)
    pltpu.sync_copy(ssum_v, ssum_out.at[rb_rel, pl.ds(s0_al, TOK_W)])
    for cp in gcopies:
        cp.wait()
    pltpu.sync_copy(vals_v, g_out.at[rb_rel, pl.ds(s0_al, TOK_W)])


def _sc_rows(xf, y_flat):
    mesh = plsc.VectorSubcoreMesh(core_axis_name="c", subcore_axis_name="s")
    f = functools.partial(
        pl.kernel,
        mesh=mesh,
        out_type=[
            jax.ShapeDtypeStruct((SC_ROWS, S), jnp.float32),
            jax.ShapeDtypeStruct((SC_ROWS, S), jnp.float32),
        ],
        scratch_types=[
            pltpu.VMEM((TOK_W,), jnp.int32),
            pltpu.VMEM((CT_W, 128), jnp.int32),
            pltpu.VMEM((TOK_W,), jnp.float32),
            pltpu.VMEM((TOK_W,), jnp.float32),
            pltpu.VMEM((CT_W * 1024,), jnp.float32),
            pltpu.VMEM((CT_W * 1024,), jnp.float32),
            pltpu.SemaphoreType.DMA,
            pltpu.SemaphoreType.DMA,
            pltpu.SemaphoreType.DMA,
        ],
    )(_sc_body)
    return f(xf, y_flat)


def _ce_tc_kernel(x_ref, y_ref, l_ref):
    xb = x_ref[0]
    y_row = y_ref[0]
    e = jnp.exp(jnp.minimum(xb, 80.0))
    ssum = jnp.sum(e, axis=0, keepdims=True)
    cids = lax.broadcasted_iota(jnp.int32, (C, S_BLK), 0)
    g = jnp.sum(jnp.where(cids == y_row, xb, 0.0), axis=0, keepdims=True)
    l_ref[0] = jnp.log(ssum) - g


def _select_kernel(l_tc_ref, ssum_ref, g_ref, out_ref):
    l_sc = jnp.log(ssum_ref[...]) - g_ref[...]
    lv = jnp.concatenate([l_tc_ref[:, 0, :], l_sc], axis=0)
    bits = lax.bitcast_convert_type(lv, jnp.int32)
    ordv = jnp.where(bits < 0, bits ^ jnp.int32(0x7FFFFFFF), bits)
    int_min = jnp.int32(-2147483648)
    p = jnp.sum((ordv >= 0).astype(jnp.int32), axis=1, keepdims=True)
    t0 = jnp.where(p >= N_KEEP, jnp.int32(0), int_min)

    def body(i, t):
        cand = t | (jnp.int32(1) << (30 - i))
        cnt = jnp.sum((ordv >= cand).astype(jnp.int32), axis=1, keepdims=True)
        return jnp.where(cnt >= N_KEEP, cand, t)

    t = lax.fori_loop(0, 31, body, t0)
    vbits = jnp.where(t < 0, t ^ jnp.int32(0x7FFFFFFF), t)
    thr = lax.bitcast_convert_type(vbits, jnp.float32)
    gt = ordv > t
    cnt_gt = jnp.sum(gt.astype(jnp.float32), axis=1, keepdims=True)
    sum_gt = jnp.sum(jnp.where(gt, lv, 0.0), axis=1, keepdims=True)
    row_total = sum_gt + (N_KEEP - cnt_gt) * thr
    out_ref[0, 0] = jnp.sum(row_total) / (N_KEEP * B)


def kernel(x, y):
    xf = (x.reshape(B, RT, 8, S // 128, 128)
          .transpose(0, 1, 3, 2, 4).reshape(-1))
    y32 = y.astype(jnp.int32)
    ssum_sc, g_sc = _sc_rows(xf, y32.reshape(-1))

    l_tc = pl.pallas_call(
        _ce_tc_kernel,
        grid=(TC_ROWS, S_GRID),
        in_specs=[
            pl.BlockSpec((1, C, S_BLK), lambda b, sb: (b, 0, sb)),
            pl.BlockSpec((1, 1, S_BLK), lambda b, sb: (b, 0, sb)),
        ],
        out_specs=pl.BlockSpec((1, 1, S_BLK), lambda b, sb: (b, 0, sb)),
        out_shape=jax.ShapeDtypeStruct((TC_ROWS, 1, S), jnp.float32),
    )(x, y32.reshape(B, 1, S))

    out = pl.pallas_call(
        _select_kernel,
        out_specs=pl.BlockSpec(memory_space=pltpu.SMEM),
        out_shape=jax.ShapeDtypeStruct((1, 1), jnp.float32),
    )(l_tc, ssum_sc, g_sc)
    return out[0, 0]


# speedup vs baseline: 1.0031x; 1.0009x over previous
import functools

import jax
import jax.numpy as jnp
from jax import lax
from jax.experimental import pallas as pl
from jax.experimental.pallas import tpu as pltpu
from jax.experimental.pallas import tpu_sc as plsc

B, C, S = 8, 1000, 8192
RATIO = 0.2
N_KEEP = int(S * RATIO)
S_BLK = 4096
S_GRID = S // S_BLK

TC_ROWS = 6
SC_ROWS = B - TC_ROWS
W_PER_ROW = 16
TOK_W = S // W_PER_ROW
CT_W = TOK_W // 128
RT = C // 8


def _sc_body(xf, y_hbm, ssum_out, g_out,
             y_v, idx_v, vals_v, ssum_v, buf_a, buf_b, sem_a, sem_b, sem_g):
    w = lax.axis_index("s") * 2 + lax.axis_index("c")
    w2 = w & 15
    rb_rel = w >> 4
    b_row = TC_ROWS + rb_rel
    s0 = w2 * TOK_W
    ct0 = w2 * CT_W

    y_off = pl.multiple_of(b_row * S + s0, TOK_W)
    pltpu.sync_copy(y_hbm.at[pl.ds(y_off, TOK_W)], y_v)
    b_off = b_row * (C * S)
    lane = lax.iota(jnp.int32, 16)
    for k in range(CT_W):
        for c8 in range(8):
            j = k * 128 + c8 * 16
            yv = y_v[pl.ds(j, 16)]
            sv = s0 + j + lane
            idx = (b_off + ((yv >> 3) << 16) + ((sv >> 7) << 10)
                   + ((yv & 7) << 7) + (sv & 127))
            idx_v[k, pl.ds(c8 * 16, 16)] = idx
    gcopies = [pltpu.async_copy(xf.at[idx_v.at[k]],
                                vals_v.at[pl.ds(k * 128, 128)], sem_g)
               for k in range(CT_W)]

    def chunk_src(c):
        off = pl.multiple_of(b_off + (c << 16) + (ct0 << 10), 1024)
        return xf.at[pl.ds(off, CT_W * 1024)]

    pltpu.async_copy(chunk_src(0), buf_a, sem_a)
    pltpu.async_copy(chunk_src(1), buf_b, sem_b)

    def accumulate(buf, acc):
        acc = list(acc)
        for r in range(8):
            for ct in range(CT_W):
                for lg in range(8):
                    k = ct * 8 + lg
                    acc[k] = acc[k] + jnp.exp(
                        buf[pl.ds(ct * 1024 + r * 128 + lg * 16, 16)])
        return acc

    def body(i2, acc):
        c0 = 2 * i2
        pltpu.make_async_copy(chunk_src(0), buf_a, sem_a).wait()

        @pl.when(c0 + 2 < RT)
        def _():
            pltpu.async_copy(chunk_src(c0 + 2), buf_a, sem_a)

        acc = accumulate(buf_a, acc)
        pltpu.make_async_copy(chunk_src(0), buf_b, sem_b).wait()

        @pl.when(c0 + 3 < RT)
        def _():
            pltpu.async_copy(chunk_src(c0 + 3), buf_b, sem_b)

        return tuple(accumulate(buf_b, acc))

    acc0 = tuple(jnp.zeros((16,), jnp.float32) for _ in range(TOK_W // 16))
    acc = lax.fori_loop(0, (RT - 1) // 2, body, acc0)
    pltpu.make_async_copy(chunk_src(0), buf_a, sem_a).wait()
    acc = accumulate(buf_a, acc)

    for k in range(TOK_W // 16):
        ssum_v[pl.ds(k * 16, 16)] = acc[k]
    s0_al = pl.multiple_of(s0, TOK_W)
    pltpu.sync_copy(ssum_v, ssum_out.at[rb_rel, pl.ds(s0_al, TOK_W)])
    for cp in gcopies:
        cp.wait()
    pltpu.sync_copy(vals_v, g_out.at[rb_rel, pl.ds(s0_al, TOK_W)])


def _sc_rows(xf, y_flat):
    mesh = plsc.VectorSubcoreMesh(core_axis_name="c", subcore_axis_name="s")
    f = functools.partial(
        pl.kernel,
        mesh=mesh,
        out_type=[
            jax.ShapeDtypeStruct((SC_ROWS, S), jnp.float32),
            jax.ShapeDtypeStruct((SC_ROWS, S), jnp.float32),
        ],
        scratch_types=[
            pltpu.VMEM((TOK_W,), jnp.int32),
            pltpu.VMEM((CT_W, 128), jnp.int32),
            pltpu.VMEM((TOK_W,), jnp.float32),
            pltpu.VMEM((TOK_W,), jnp.float32),
            pltpu.VMEM((CT_W * 1024,), jnp.float32),
            pltpu.VMEM((CT_W * 1024,), jnp.float32),
            pltpu.SemaphoreType.DMA,
            pltpu.SemaphoreType.DMA,
            pltpu.SemaphoreType.DMA,
        ],
    )(_sc_body)
    return f(xf, y_flat)


def _ce_tc_kernel(x_ref, y_ref, l_ref):
    xb = x_ref[0]
    y_row = y_ref[0]
    e = jnp.exp(jnp.minimum(xb, 80.0))
    ssum = jnp.sum(e, axis=0, keepdims=True)
    cids = lax.broadcasted_iota(jnp.int32, (C, S_BLK), 0)
    g = jnp.sum(jnp.where(cids == y_row, xb, 0.0), axis=0, keepdims=True)
    l_ref[0] = jnp.log(ssum) - g


def _select_kernel(l_tc_ref, ssum_ref, g_ref, out_ref):
    l_sc = jnp.log(ssum_ref[...]) - g_ref[...]
    lv = jnp.concatenate([l_tc_ref[:, 0, :], l_sc], axis=0)
    bits = lax.bitcast_convert_type(lv, jnp.int32)
    ordv = jnp.where(bits < 0, bits ^ jnp.int32(0x7FFFFFFF), bits)
    int_min = jnp.int32(-2147483648)
    p = jnp.sum((ordv >= 0).astype(jnp.int32), axis=1, keepdims=True)
    t0 = jnp.where(p >= N_KEEP, jnp.int32(0), int_min)

    def body(i, t):
        cand = t | (jnp.int32(1) << (30 - i))
        cnt = jnp.sum((ordv >= cand).astype(jnp.int32), axis=1, keepdims=True)
        return jnp.where(cnt >= N_KEEP, cand, t)

    t = lax.fori_loop(0, 31, body, t0)
    vbits = jnp.where(t < 0, t ^ jnp.int32(0x7FFFFFFF), t)
    thr = lax.bitcast_convert_type(vbits, jnp.float32)
    gt = ordv > t
    cnt_gt = jnp.sum(gt.astype(jnp.float32), axis=1, keepdims=True)
    sum_gt = jnp.sum(jnp.where(gt, lv, 0.0), axis=1, keepdims=True)
    row_total = sum_gt + (N_KEEP - cnt_gt) * thr
    out_ref[0, 0] = jnp.sum(row_total) / (N_KEEP * B)


def kernel(x, y):
    xf = (x.reshape(B, RT, 8, S // 128, 128)
          .transpose(0, 1, 3, 2, 4).reshape(-1))
    y32 = y.astype(jnp.int32)
    ssum_sc, g_sc = _sc_rows(xf, y32.reshape(-1))

    l_tc = pl.pallas_call(
        _ce_tc_kernel,
        grid=(TC_ROWS, S_GRID),
        in_specs=[
            pl.BlockSpec((1, C, S_BLK), lambda b, sb: (b, 0, sb)),
            pl.BlockSpec((1, 1, S_BLK), lambda b, sb: (b, 0, sb)),
        ],
        out_specs=pl.BlockSpec((1, 1, S_BLK), lambda b, sb: (b, 0, sb)),
        out_shape=jax.ShapeDtypeStruct((TC_ROWS, 1, S), jnp.float32),
    )(x, y32.reshape(B, 1, S))

    out = pl.pallas_call(
        _select_kernel,
        out_specs=pl.BlockSpec(memory_space=pltpu.SMEM),
        out_shape=jax.ShapeDtypeStruct((1, 1), jnp.float32),
    )(l_tc, ssum_sc, g_sc)
    return out[0, 0]


# speedup vs baseline: 1.0061x; 1.0030x over previous
import functools

import jax
import jax.numpy as jnp
from jax import lax
from jax.experimental import pallas as pl
from jax.experimental.pallas import tpu as pltpu
from jax.experimental.pallas import tpu_sc as plsc

B, C, S = 8, 1000, 8192
RATIO = 0.2
N_KEEP = int(S * RATIO)
S_BLK = 4096
S_GRID = S // S_BLK

TC_ROWS = 6
SC_ROWS = B - TC_ROWS
W_PER_ROW = 16
TOK_W = S // W_PER_ROW
CT_W = TOK_W // 128
RT = C // 8


def _sc_body(xf, y_hbm, ssum_out, g_out,
             y_v, idx_v, vals_v, ssum_v, buf_a, buf_b, sem_a, sem_b, sem_g):
    w = lax.axis_index("s") * 2 + lax.axis_index("c")
    w2 = w & 15
    rb_rel = w >> 4
    b_row = TC_ROWS + rb_rel
    s0 = w2 * TOK_W
    ct0 = w2 * CT_W

    y_off = pl.multiple_of(b_row * S + s0, TOK_W)
    pltpu.sync_copy(y_hbm.at[pl.ds(y_off, TOK_W)], y_v)
    b_off = b_row * (C * S)
    lane = lax.iota(jnp.int32, 16)
    for k in range(CT_W):
        for c8 in range(8):
            j = k * 128 + c8 * 16
            yv = y_v[pl.ds(j, 16)]
            sv = s0 + j + lane
            idx = (b_off + ((yv >> 3) << 16) + ((sv >> 7) << 10)
                   + ((yv & 7) << 7) + (sv & 127))
            idx_v[k, pl.ds(c8 * 16, 16)] = idx
    gcopies = [pltpu.async_copy(xf.at[idx_v.at[k]],
                                vals_v.at[pl.ds(k * 128, 128)], sem_g)
               for k in range(CT_W)]

    G = 5
    NGRP = RT // G

    def issue_group(g, buf, sem):
        for j in range(G):
            off = pl.multiple_of(b_off + ((g * G + j) << 16) + (ct0 << 10),
                                 1024)
            pltpu.async_copy(xf.at[pl.ds(off, CT_W * 1024)],
                             buf.at[pl.ds(j * CT_W * 1024, CT_W * 1024)],
                             sem)

    def wait_group(buf, sem):
        pltpu.make_async_copy(xf.at[pl.ds(0, G * CT_W * 1024)], buf,
                              sem).wait()

    def accumulate(buf, acc):
        def rt_body(i, acc):
            base = i * (CT_W * 1024)
            acc = list(acc)
            for r in range(8):
                for ct in range(CT_W):
                    for lg in range(8):
                        k = ct * 8 + lg
                        acc[k] = acc[k] + jnp.exp(
                            buf[pl.ds(base + ct * 1024 + r * 128 + lg * 16,
                                      16)])
            return tuple(acc)

        return lax.fori_loop(0, G, rt_body, tuple(acc))

    issue_group(0, buf_a, sem_a)
    issue_group(1, buf_b, sem_b)

    def body(i2, acc):
        g0 = 2 * i2
        wait_group(buf_a, sem_a)

        @pl.when(g0 + 2 < NGRP)
        def _():
            issue_group(g0 + 2, buf_a, sem_a)

        acc = accumulate(buf_a, acc)
        wait_group(buf_b, sem_b)

        @pl.when(g0 + 3 < NGRP)
        def _():
            issue_group(g0 + 3, buf_b, sem_b)

        return accumulate(buf_b, acc)

    acc0 = tuple(jnp.zeros((16,), jnp.float32) for _ in range(TOK_W // 16))
    acc = lax.fori_loop(0, (NGRP - 1) // 2, body, acc0)
    wait_group(buf_a, sem_a)
    acc = accumulate(buf_a, acc)

    for k in range(TOK_W // 16):
        ssum_v[pl.ds(k * 16, 16)] = acc[k]
    s0_al = pl.multiple_of(s0, TOK_W)
    pltpu.sync_copy(ssum_v, ssum_out.at[rb_rel, pl.ds(s0_al, TOK_W)])
    for cp in gcopies:
        cp.wait()
    pltpu.sync_copy(vals_v, g_out.at[rb_rel, pl.ds(s0_al, TOK_W)])


def _sc_rows(xf, y_flat):
    mesh = plsc.VectorSubcoreMesh(core_axis_name="c", subcore_axis_name="s")
    f = functools.partial(
        pl.kernel,
        mesh=mesh,
        out_type=[
            jax.ShapeDtypeStruct((SC_ROWS, S), jnp.float32),
            jax.ShapeDtypeStruct((SC_ROWS, S), jnp.float32),
        ],
        scratch_types=[
            pltpu.VMEM((TOK_W,), jnp.int32),
            pltpu.VMEM((CT_W, 128), jnp.int32),
            pltpu.VMEM((TOK_W,), jnp.float32),
            pltpu.VMEM((TOK_W,), jnp.float32),
            pltpu.VMEM((5 * CT_W * 1024,), jnp.float32),
            pltpu.VMEM((5 * CT_W * 1024,), jnp.float32),
            pltpu.SemaphoreType.DMA,
            pltpu.SemaphoreType.DMA,
            pltpu.SemaphoreType.DMA,
        ],
    )(_sc_body)
    return f(xf, y_flat)


def _ce_tc_kernel(x_ref, y_ref, l_ref):
    xb = x_ref[0]
    y_row = y_ref[0]
    e = jnp.exp(jnp.minimum(xb, 80.0))
    ssum = jnp.sum(e, axis=0, keepdims=True)
    cids = lax.broadcasted_iota(jnp.int32, (C, S_BLK), 0)
    g = jnp.sum(jnp.where(cids == y_row, xb, 0.0), axis=0, keepdims=True)
    l_ref[0] = jnp.log(ssum) - g


def _select_kernel(l_tc_ref, ssum_ref, g_ref, out_ref):
    l_sc = jnp.log(ssum_ref[...]) - g_ref[...]
    lv = jnp.concatenate([l_tc_ref[:, 0, :], l_sc], axis=0)
    bits = lax.bitcast_convert_type(lv, jnp.int32)
    ordv = jnp.where(bits < 0, bits ^ jnp.int32(0x7FFFFFFF), bits)
    int_min = jnp.int32(-2147483648)
    p = jnp.sum((ordv >= 0).astype(jnp.int32), axis=1, keepdims=True)
    t0 = jnp.where(p >= N_KEEP, jnp.int32(0), int_min)

    def body(i, t):
        cand = t | (jnp.int32(1) << (30 - i))
        cnt = jnp.sum((ordv >= cand).astype(jnp.int32), axis=1, keepdims=True)
        return jnp.where(cnt >= N_KEEP, cand, t)

    t = lax.fori_loop(0, 31, body, t0)
    vbits = jnp.where(t < 0, t ^ jnp.int32(0x7FFFFFFF), t)
    thr = lax.bitcast_convert_type(vbits, jnp.float32)
    gt = ordv > t
    cnt_gt = jnp.sum(gt.astype(jnp.float32), axis=1, keepdims=True)
    sum_gt = jnp.sum(jnp.where(gt, lv, 0.0), axis=1, keepdims=True)
    row_total = sum_gt + (N_KEEP - cnt_gt) * thr
    out_ref[0, 0] = jnp.sum(row_total) / (N_KEEP * B)


def kernel(x, y):
    xf = (x.reshape(B, RT, 8, S // 128, 128)
          .transpose(0, 1, 3, 2, 4).reshape(-1))
    y32 = y.astype(jnp.int32)
    ssum_sc, g_sc = _sc_rows(xf, y32.reshape(-1))

    l_tc = pl.pallas_call(
        _ce_tc_kernel,
        grid=(TC_ROWS, S_GRID),
        in_specs=[
            pl.BlockSpec((1, C, S_BLK), lambda b, sb: (b, 0, sb)),
            pl.BlockSpec((1, 1, S_BLK), lambda b, sb: (b, 0, sb)),
        ],
        out_specs=pl.BlockSpec((1, 1, S_BLK), lambda b, sb: (b, 0, sb)),
        out_shape=jax.ShapeDtypeStruct((TC_ROWS, 1, S), jnp.float32),
    )(x, y32.reshape(B, 1, S))

    out = pl.pallas_call(
        _select_kernel,
        out_specs=pl.BlockSpec(memory_space=pltpu.SMEM),
        out_shape=jax.ShapeDtypeStruct((1, 1), jnp.float32),
    )(l_tc, ssum_sc, g_sc)
    return out[0, 0]
